# Initial kernel scaffold; baseline (speedup 1.0000x reference)
import jax
import jax.numpy as jnp
from jax.experimental import pallas as pl


def kernel(values, row_idx, col_idx, W, bias):
    raise NotImplementedError("write your pallas kernel here")


# speedup vs baseline: 2.2945x; 2.2945x over previous
import functools

import jax
import jax.numpy as jnp
from jax import lax
from jax.experimental import pallas as pl
from jax.experimental.pallas import tpu as pltpu
from jax.experimental.pallas import tpu_sc as plsc

N_SEG = 10000
N_SEG_P = 10240
NNZ = 320000
D = 128
NC = 2
NS = 16
PER_TILE = NNZ // NS
SUB = 80
SCHUNK = 160
N_SCHUNK = PER_TILE // SCHUNK
S_NSUB = SCHUNK // SUB
GCHUNK = 800
N_GCHUNK = PER_TILE // GCHUNK
G_NSUB = GCHUNK // SUB
ROWS_PT = N_SEG_P // NS


@functools.cache
def _mesh():
    return plsc.VectorSubcoreMesh(
        core_axis_name="c", subcore_axis_name="s", num_cores=NC, num_subcores=NS)


def _seg_sums_body(values, idxflat, zsum, sums,
                   vals_v, idx_vs, sh_sum):
    c = lax.axis_index("c")
    s = lax.axis_index("s")
    pltpu.sync_copy(zsum.at[pl.ds(s * ROWS_PT, ROWS_PT)],
                    sh_sum.at[pl.ds(s * ROWS_PT, ROWS_PT)])
    plsc.subcore_barrier()

    @pl.loop(0, N_SCHUNK)
    def _chunk(i):
        base = s * PER_TILE + i * SCHUNK
        for j in range(S_NSUB):
            pltpu.sync_copy(idxflat.at[pl.ds(c * NNZ + base + j * SUB, SUB)],
                            idx_vs[j])
        pltpu.sync_copy(values.at[pl.ds(base, SCHUNK)], vals_v)
        for j in range(S_NSUB):
            pltpu.sync_copy(vals_v.at[pl.ds(j * SUB, SUB)],
                            sh_sum.at[idx_vs[j]], add=True)

    plsc.subcore_barrier()
    pltpu.sync_copy(sh_sum.at[pl.ds(s * ROWS_PT, ROWS_PT)],
                    sums.at[c, pl.ds(s * ROWS_PT, ROWS_PT)])


@functools.cache
def _seg_sums():
    return pl.kernel(
        _seg_sums_body,
        out_type=jax.ShapeDtypeStruct((NC, N_SEG_P, D), jnp.float32),
        mesh=_mesh(),
        scratch_types=[pltpu.VMEM((SCHUNK, D), jnp.float32),
                       [pltpu.VMEM((SUB,), jnp.int32) for _ in range(S_NSUB)],
                       pltpu.VMEM_SHARED((N_SEG_P, D), jnp.float32)],
    )


def _seg_cnts_body(idxflat, zsum, ones_in, cnts,
                   ones_v, idx_vs, sh_cnt):
    c = lax.axis_index("c")
    s = lax.axis_index("s")
    pltpu.sync_copy(ones_in, ones_v)
    pltpu.sync_copy(zsum.at[pl.ds(s * ROWS_PT, ROWS_PT)],
                    sh_cnt.at[pl.ds(s * ROWS_PT, ROWS_PT)])
    plsc.subcore_barrier()

    @pl.loop(0, N_SCHUNK)
    def _chunk(i):
        base = s * PER_TILE + i * SCHUNK
        for j in range(S_NSUB):
            pltpu.sync_copy(idxflat.at[pl.ds(c * NNZ + base + j * SUB, SUB)],
                            idx_vs[j])
        for j in range(S_NSUB):
            pltpu.sync_copy(ones_v, sh_cnt.at[idx_vs[j]], add=True)

    plsc.subcore_barrier()
    pltpu.sync_copy(sh_cnt.at[pl.ds(s * ROWS_PT, ROWS_PT)],
                    cnts.at[c, pl.ds(s * ROWS_PT, ROWS_PT)])


@functools.cache
def _seg_cnts():
    return pl.kernel(
        _seg_cnts_body,
        out_type=jax.ShapeDtypeStruct((NC, N_SEG_P, D), jnp.float32),
        mesh=_mesh(),
        scratch_types=[pltpu.VMEM((SUB, D), jnp.float32),
                       [pltpu.VMEM((SUB,), jnp.int32) for _ in range(S_NSUB)],
                       pltpu.VMEM_SHARED((N_SEG_P, D), jnp.float32)],
    )


def _gather_body(tabflat, idxflatb, gg, g_v, idx_vs):
    c = lax.axis_index("c")
    s = lax.axis_index("s")

    @pl.loop(0, N_GCHUNK)
    def _chunk(i):
        base = s * PER_TILE + i * GCHUNK
        for j in range(G_NSUB):
            pltpu.sync_copy(idxflatb.at[pl.ds(c * NNZ + base + j * SUB, SUB)],
                            idx_vs[j])
        for j in range(G_NSUB):
            pltpu.sync_copy(tabflat.at[idx_vs[j]],
                            g_v.at[pl.ds(j * SUB, SUB)])
        pltpu.sync_copy(g_v, gg.at[c, pl.ds(base, GCHUNK)])


@functools.cache
def _gather():
    return pl.kernel(
        _gather_body,
        out_type=jax.ShapeDtypeStruct((NC, NNZ, D), jnp.float32),
        mesh=_mesh(),
        scratch_types=[pltpu.VMEM((GCHUNK, D), jnp.float32),
                       [pltpu.VMEM((SUB,), jnp.int32) for _ in range(G_NSUB)]],
    )


def _tables_kernel(sums_ref, cnts_ref, w_ref, bias_ref, tab_ref):
    cnt_r = jnp.maximum(cnts_ref[0, :, 0:1], 1.0)
    cnt_c = jnp.maximum(cnts_ref[1, :, 0:1], 1.0)
    pooled_r = sums_ref[0] / cnt_r
    pooled_c = sums_ref[1] / cnt_c
    g = jnp.sum(sums_ref[0], axis=0, keepdims=True) * (1.0 / NNZ)
    gw = jnp.dot(g, w_ref[3], preferred_element_type=jnp.float32) + bias_ref[...]
    tab_ref[0] = jnp.dot(pooled_r, w_ref[1],
                         preferred_element_type=jnp.float32) + gw
    tab_ref[1] = jnp.dot(pooled_c, w_ref[2],
                         preferred_element_type=jnp.float32)


def _tables(sums, cnts, W, bias):
    return pl.pallas_call(
        _tables_kernel,
        out_shape=jax.ShapeDtypeStruct((NC, N_SEG_P, D), jnp.float32),
    )(sums, cnts, W, bias)


_FBLK = 2000


def _final_kernel(v_ref, g0_ref, g1_ref, w0_ref, out_ref):
    out_ref[...] = (jnp.dot(v_ref[...], w0_ref[...],
                            preferred_element_type=jnp.float32)
                    + g0_ref[0] + g1_ref[0])


def _final(values, gg, w0):
    return pl.pallas_call(
        _final_kernel,
        grid=(NNZ // _FBLK,),
        in_specs=[pl.BlockSpec((_FBLK, D), lambda i: (i, 0)),
                  pl.BlockSpec((1, _FBLK, D), lambda i: (0, i, 0)),
                  pl.BlockSpec((1, _FBLK, D), lambda i: (1, i, 0)),
                  pl.BlockSpec((D, D), lambda i: (0, 0))],
        out_specs=pl.BlockSpec((_FBLK, D), lambda i: (i, 0)),
        out_shape=jax.ShapeDtypeStruct((NNZ, D), jnp.float32),
    )(values, gg, gg, w0)


def kernel(values, row_idx, col_idx, W, bias):
    idxflat = jnp.concatenate([row_idx, col_idx])
    idxflatb = jnp.concatenate([row_idx, col_idx + N_SEG_P])
    zsum = jnp.zeros((N_SEG_P, D), jnp.float32)
    ones_in = jnp.ones((SUB, D), jnp.float32)
    sums = _seg_sums()(values, idxflat, zsum)
    cnts = _seg_cnts()(idxflat, zsum, ones_in)
    tabs = _tables(sums, cnts, W, bias)
    gg = _gather()(tabs.reshape(NC * N_SEG_P, D), idxflatb)
    return _final(values, gg, W[0])


# speedup vs baseline: 3.3036x; 1.4398x over previous
import functools

import jax
import jax.numpy as jnp
from jax import lax
from jax.experimental import pallas as pl
from jax.experimental.pallas import tpu as pltpu
from jax.experimental.pallas import tpu_sc as plsc

N_SEG = 10000
N_SEG_P = 10240
NNZ = 320000
D = 128
NC = 2
NS = 16
PER_TILE = NNZ // NS
SUB = 80
SCHUNK = 160
N_SCHUNK = PER_TILE // SCHUNK
S_NSUB = SCHUNK // SUB
GCHUNK = 800
N_GCHUNK = PER_TILE // GCHUNK
G_NSUB = GCHUNK // SUB
ROWS_PT = N_SEG_P // NS


@functools.cache
def _mesh():
    return plsc.VectorSubcoreMesh(
        core_axis_name="c", subcore_axis_name="s", num_cores=NC, num_subcores=NS)


def _seg_sums_body(values, idxflat, zsum, sums,
                   vals_v, idx_vs, sh_sum, sem):
    c = lax.axis_index("c")
    s = lax.axis_index("s")
    pltpu.sync_copy(zsum.at[pl.ds(s * ROWS_PT, ROWS_PT)],
                    sh_sum.at[pl.ds(s * ROWS_PT, ROWS_PT)])
    plsc.subcore_barrier()

    @pl.loop(0, N_SCHUNK)
    def _chunk(i):
        base = s * PER_TILE + i * SCHUNK
        loads = [pltpu.async_copy(
                     idxflat.at[pl.ds(c * NNZ + base + j * SUB, SUB)],
                     idx_vs[j], sem) for j in range(S_NSUB)]
        loads.append(pltpu.async_copy(values.at[pl.ds(base, SCHUNK)],
                                      vals_v, sem))
        for dsc in loads:
            dsc.wait()
        adds = [pltpu.async_copy(vals_v.at[pl.ds(j * SUB, SUB)],
                                 sh_sum.at[idx_vs[j]], sem, add=True)
                for j in range(S_NSUB)]
        for dsc in adds:
            dsc.wait()

    plsc.subcore_barrier()
    pltpu.sync_copy(sh_sum.at[pl.ds(s * ROWS_PT, ROWS_PT)],
                    sums.at[c, pl.ds(s * ROWS_PT, ROWS_PT)])


@functools.cache
def _seg_sums():
    return pl.kernel(
        _seg_sums_body,
        out_type=jax.ShapeDtypeStruct((NC, N_SEG_P, D), jnp.float32),
        mesh=_mesh(),
        scratch_types=[pltpu.VMEM((SCHUNK, D), jnp.float32),
                       [pltpu.VMEM((SUB,), jnp.int32) for _ in range(S_NSUB)],
                       pltpu.VMEM_SHARED((N_SEG_P, D), jnp.float32),
                       pltpu.SemaphoreType.DMA],
    )


def _seg_cnts_body(idxflat, zsum, ones_in, cnts,
                   ones_v, idx_vs, sh_cnt, sem):
    c = lax.axis_index("c")
    s = lax.axis_index("s")
    pltpu.sync_copy(ones_in, ones_v)
    pltpu.sync_copy(zsum.at[pl.ds(s * ROWS_PT, ROWS_PT)],
                    sh_cnt.at[pl.ds(s * ROWS_PT, ROWS_PT)])
    plsc.subcore_barrier()

    @pl.loop(0, N_SCHUNK)
    def _chunk(i):
        base = s * PER_TILE + i * SCHUNK
        loads = [pltpu.async_copy(
                     idxflat.at[pl.ds(c * NNZ + base + j * SUB, SUB)],
                     idx_vs[j], sem) for j in range(S_NSUB)]
        for dsc in loads:
            dsc.wait()
        adds = [pltpu.async_copy(ones_v, sh_cnt.at[idx_vs[j]], sem, add=True)
                for j in range(S_NSUB)]
        for dsc in adds:
            dsc.wait()

    plsc.subcore_barrier()
    pltpu.sync_copy(sh_cnt.at[pl.ds(s * ROWS_PT, ROWS_PT)],
                    cnts.at[c, pl.ds(s * ROWS_PT, ROWS_PT)])


@functools.cache
def _seg_cnts():
    return pl.kernel(
        _seg_cnts_body,
        out_type=jax.ShapeDtypeStruct((NC, N_SEG_P, D), jnp.float32),
        mesh=_mesh(),
        scratch_types=[pltpu.VMEM((SUB, D), jnp.float32),
                       [pltpu.VMEM((SUB,), jnp.int32) for _ in range(S_NSUB)],
                       pltpu.VMEM_SHARED((N_SEG_P, D), jnp.float32),
                       pltpu.SemaphoreType.DMA],
    )


def _gather_body(tabflat, idxflatb, gg, g_v, idx_vs, sem):
    c = lax.axis_index("c")
    s = lax.axis_index("s")

    @pl.loop(0, N_GCHUNK)
    def _chunk(i):
        base = s * PER_TILE + i * GCHUNK
        loads = [pltpu.async_copy(
                     idxflatb.at[pl.ds(c * NNZ + base + j * SUB, SUB)],
                     idx_vs[j], sem) for j in range(G_NSUB)]
        for dsc in loads:
            dsc.wait()
        gats = [pltpu.async_copy(tabflat.at[idx_vs[j]],
                                 g_v.at[pl.ds(j * SUB, SUB)], sem)
                for j in range(G_NSUB)]
        for dsc in gats:
            dsc.wait()
        pltpu.sync_copy(g_v, gg.at[c, pl.ds(base, GCHUNK)])


@functools.cache
def _gather():
    return pl.kernel(
        _gather_body,
        out_type=jax.ShapeDtypeStruct((NC, NNZ, D), jnp.float32),
        mesh=_mesh(),
        scratch_types=[pltpu.VMEM((GCHUNK, D), jnp.float32),
                       [pltpu.VMEM((SUB,), jnp.int32) for _ in range(G_NSUB)],
                       pltpu.SemaphoreType.DMA],
    )


def _tables_kernel(sums_ref, cnts_ref, w_ref, bias_ref, tab_ref):
    cnt_r = jnp.maximum(cnts_ref[0, :, 0:1], 1.0)
    cnt_c = jnp.maximum(cnts_ref[1, :, 0:1], 1.0)
    pooled_r = sums_ref[0] / cnt_r
    pooled_c = sums_ref[1] / cnt_c
    g = jnp.sum(sums_ref[0], axis=0, keepdims=True) * (1.0 / NNZ)
    gw = jnp.dot(g, w_ref[3], preferred_element_type=jnp.float32) + bias_ref[...]
    tab_ref[0] = jnp.dot(pooled_r, w_ref[1],
                         preferred_element_type=jnp.float32) + gw
    tab_ref[1] = jnp.dot(pooled_c, w_ref[2],
                         preferred_element_type=jnp.float32)


def _tables(sums, cnts, W, bias):
    return pl.pallas_call(
        _tables_kernel,
        out_shape=jax.ShapeDtypeStruct((NC, N_SEG_P, D), jnp.float32),
    )(sums, cnts, W, bias)


_FBLK = 2000


def _final_kernel(v_ref, g0_ref, g1_ref, w0_ref, out_ref):
    out_ref[...] = (jnp.dot(v_ref[...], w0_ref[...],
                            preferred_element_type=jnp.float32)
                    + g0_ref[0] + g1_ref[0])


def _final(values, gg, w0):
    return pl.pallas_call(
        _final_kernel,
        grid=(NNZ // _FBLK,),
        in_specs=[pl.BlockSpec((_FBLK, D), lambda i: (i, 0)),
                  pl.BlockSpec((1, _FBLK, D), lambda i: (0, i, 0)),
                  pl.BlockSpec((1, _FBLK, D), lambda i: (1, i, 0)),
                  pl.BlockSpec((D, D), lambda i: (0, 0))],
        out_specs=pl.BlockSpec((_FBLK, D), lambda i: (i, 0)),
        out_shape=jax.ShapeDtypeStruct((NNZ, D), jnp.float32),
    )(values, gg, gg, w0)


def kernel(values, row_idx, col_idx, W, bias):
    idxflat = jnp.concatenate([row_idx, col_idx])
    idxflatb = jnp.concatenate([row_idx, col_idx + N_SEG_P])
    zsum = jnp.zeros((N_SEG_P, D), jnp.float32)
    ones_in = jnp.ones((SUB, D), jnp.float32)
    sums = _seg_sums()(values, idxflat, zsum)
    cnts = _seg_cnts()(idxflat, zsum, ones_in)
    tabs = _tables(sums, cnts, W, bias)
    gg = _gather()(tabs.reshape(NC * N_SEG_P, D), idxflatb)
    return _final(values, gg, W[0])


# speedup vs baseline: 3.4710x; 1.0507x over previous
import functools

import jax
import jax.numpy as jnp
from jax import lax
from jax.experimental import pallas as pl
from jax.experimental.pallas import tpu as pltpu
from jax.experimental.pallas import tpu_sc as plsc

N_SEG = 10000
N_SEG_P = 10240
NNZ = 320000
D = 128
NC = 2
NS = 16
PER_TILE = NNZ // NS
SUB = 80
SCHUNK = 160
N_SCHUNK = PER_TILE // SCHUNK
S_NSUB = SCHUNK // SUB
GCHUNK = 400
N_GCHUNK = PER_TILE // GCHUNK
G_NSUB = GCHUNK // SUB
ROWS_PT = N_SEG_P // NS


@functools.cache
def _mesh():
    return plsc.VectorSubcoreMesh(
        core_axis_name="c", subcore_axis_name="s", num_cores=NC, num_subcores=NS)


def _seg_sums_body(values, idxflat, zsum, sums,
                   vals_vs, idx_vs, sh_sum, sems):
    c = lax.axis_index("c")
    s = lax.axis_index("s")
    pltpu.sync_copy(zsum.at[pl.ds(s * ROWS_PT, ROWS_PT)],
                    sh_sum.at[pl.ds(s * ROWS_PT, ROWS_PT)])
    plsc.subcore_barrier()

    def fire_loads(i, b):
        base = s * PER_TILE + i * SCHUNK
        loads = [pltpu.async_copy(
                     idxflat.at[pl.ds(c * NNZ + base + j * SUB, SUB)],
                     idx_vs[b][j], sems[b]) for j in range(S_NSUB)]
        loads.append(pltpu.async_copy(values.at[pl.ds(base, SCHUNK)],
                                      vals_vs[b], sems[b]))
        return loads

    def fire_adds(b):
        return [pltpu.async_copy(vals_vs[b].at[pl.ds(j * SUB, SUB)],
                                 sh_sum.at[idx_vs[b][j]], sems[b], add=True)
                for j in range(S_NSUB)]

    @pl.loop(0, N_SCHUNK - 1, step=2)
    def _pair(i):
        l0 = fire_loads(i, 0)
        for dsc in l0:
            dsc.wait()
        a0 = fire_adds(0)
        l1 = fire_loads(i + 1, 1)
        for dsc in a0:
            dsc.wait()
        for dsc in l1:
            dsc.wait()
        a1 = fire_adds(1)
        for dsc in a1:
            dsc.wait()

    lt = fire_loads(N_SCHUNK - 1, 0)
    for dsc in lt:
        dsc.wait()
    at = fire_adds(0)
    for dsc in at:
        dsc.wait()

    plsc.subcore_barrier()
    pltpu.sync_copy(sh_sum.at[pl.ds(s * ROWS_PT, ROWS_PT)],
                    sums.at[c, pl.ds(s * ROWS_PT, ROWS_PT)])


@functools.cache
def _seg_sums():
    return pl.kernel(
        _seg_sums_body,
        out_type=jax.ShapeDtypeStruct((NC, N_SEG_P, D), jnp.float32),
        mesh=_mesh(),
        scratch_types=[[pltpu.VMEM((SCHUNK, D), jnp.float32) for _ in range(2)],
                       [[pltpu.VMEM((SUB,), jnp.int32) for _ in range(S_NSUB)]
                        for _ in range(2)],
                       pltpu.VMEM_SHARED((N_SEG_P, D), jnp.float32),
                       [pltpu.SemaphoreType.DMA for _ in range(2)]],
    )


def _seg_cnts_body(idxflat, zsum, ones_in, cnts,
                   ones_v, idx_vs, sh_cnt, sems):
    c = lax.axis_index("c")
    s = lax.axis_index("s")
    pltpu.sync_copy(ones_in, ones_v)
    pltpu.sync_copy(zsum.at[pl.ds(s * ROWS_PT, ROWS_PT)],
                    sh_cnt.at[pl.ds(s * ROWS_PT, ROWS_PT)])
    plsc.subcore_barrier()

    def fire_loads(i, b):
        base = s * PER_TILE + i * SCHUNK
        return [pltpu.async_copy(
                    idxflat.at[pl.ds(c * NNZ + base + j * SUB, SUB)],
                    idx_vs[b][j], sems[b]) for j in range(S_NSUB)]

    def fire_adds(b):
        return [pltpu.async_copy(ones_v, sh_cnt.at[idx_vs[b][j]], sems[b],
                                 add=True) for j in range(S_NSUB)]

    @pl.loop(0, N_SCHUNK - 1, step=2)
    def _pair(i):
        l0 = fire_loads(i, 0)
        for dsc in l0:
            dsc.wait()
        a0 = fire_adds(0)
        l1 = fire_loads(i + 1, 1)
        for dsc in a0:
            dsc.wait()
        for dsc in l1:
            dsc.wait()
        a1 = fire_adds(1)
        for dsc in a1:
            dsc.wait()

    lt = fire_loads(N_SCHUNK - 1, 0)
    for dsc in lt:
        dsc.wait()
    at = fire_adds(0)
    for dsc in at:
        dsc.wait()

    plsc.subcore_barrier()
    pltpu.sync_copy(sh_cnt.at[pl.ds(s * ROWS_PT, ROWS_PT)],
                    cnts.at[c, pl.ds(s * ROWS_PT, ROWS_PT)])


@functools.cache
def _seg_cnts():
    return pl.kernel(
        _seg_cnts_body,
        out_type=jax.ShapeDtypeStruct((NC, N_SEG_P, D), jnp.float32),
        mesh=_mesh(),
        scratch_types=[pltpu.VMEM((SUB, D), jnp.float32),
                       [[pltpu.VMEM((SUB,), jnp.int32) for _ in range(S_NSUB)]
                        for _ in range(2)],
                       pltpu.VMEM_SHARED((N_SEG_P, D), jnp.float32),
                       [pltpu.SemaphoreType.DMA for _ in range(2)]],
    )


def _gather_body(tabflat, idxflatb, gg, g_vs, idx_vs, sems):
    c = lax.axis_index("c")
    s = lax.axis_index("s")

    def fire_loads(i, b):
        base = s * PER_TILE + i * GCHUNK
        return [pltpu.async_copy(
                    idxflatb.at[pl.ds(c * NNZ + base + j * SUB, SUB)],
                    idx_vs[b][j], sems[b]) for j in range(G_NSUB)]

    def fire_gats(b):
        return [pltpu.async_copy(tabflat.at[idx_vs[b][j]],
                                 g_vs[b].at[pl.ds(j * SUB, SUB)], sems[b])
                for j in range(G_NSUB)]

    def fire_store(i, b):
        base = s * PER_TILE + i * GCHUNK
        return pltpu.async_copy(g_vs[b], gg.at[c, pl.ds(base, GCHUNK)],
                                sems[b])

    @pl.loop(0, N_GCHUNK, step=2)
    def _pair(i):
        l0 = fire_loads(i, 0)
        for dsc in l0:
            dsc.wait()
        g0 = fire_gats(0)
        for dsc in g0:
            dsc.wait()
        st0 = fire_store(i, 0)
        l1 = fire_loads(i + 1, 1)
        for dsc in l1:
            dsc.wait()
        g1 = fire_gats(1)
        for dsc in g1:
            dsc.wait()
        st1 = fire_store(i + 1, 1)
        st0.wait()
        st1.wait()


@functools.cache
def _gather():
    return pl.kernel(
        _gather_body,
        out_type=jax.ShapeDtypeStruct((NC, NNZ, D), jnp.float32),
        mesh=_mesh(),
        scratch_types=[[pltpu.VMEM((GCHUNK, D), jnp.float32) for _ in range(2)],
                       [[pltpu.VMEM((SUB,), jnp.int32) for _ in range(G_NSUB)]
                        for _ in range(2)],
                       [pltpu.SemaphoreType.DMA for _ in range(2)]],
    )


def _tables_kernel(sums_ref, cnts_ref, w_ref, bias_ref, tab_ref):
    cnt_r = jnp.maximum(cnts_ref[0, :, 0:1], 1.0)
    cnt_c = jnp.maximum(cnts_ref[1, :, 0:1], 1.0)
    pooled_r = sums_ref[0] / cnt_r
    pooled_c = sums_ref[1] / cnt_c
    g = jnp.sum(sums_ref[0], axis=0, keepdims=True) * (1.0 / NNZ)
    gw = jnp.dot(g, w_ref[3], preferred_element_type=jnp.float32) + bias_ref[...]
    tab_ref[0] = jnp.dot(pooled_r, w_ref[1],
                         preferred_element_type=jnp.float32) + gw
    tab_ref[1] = jnp.dot(pooled_c, w_ref[2],
                         preferred_element_type=jnp.float32)


def _tables(sums, cnts, W, bias):
    return pl.pallas_call(
        _tables_kernel,
        out_shape=jax.ShapeDtypeStruct((NC, N_SEG_P, D), jnp.float32),
    )(sums, cnts, W, bias)


_FBLK = 2000


def _final_kernel(v_ref, g0_ref, g1_ref, w0_ref, out_ref):
    out_ref[...] = (jnp.dot(v_ref[...], w0_ref[...],
                            preferred_element_type=jnp.float32)
                    + g0_ref[0] + g1_ref[0])


def _final(values, gg, w0):
    return pl.pallas_call(
        _final_kernel,
        grid=(NNZ // _FBLK,),
        in_specs=[pl.BlockSpec((_FBLK, D), lambda i: (i, 0)),
                  pl.BlockSpec((1, _FBLK, D), lambda i: (0, i, 0)),
                  pl.BlockSpec((1, _FBLK, D), lambda i: (1, i, 0)),
                  pl.BlockSpec((D, D), lambda i: (0, 0))],
        out_specs=pl.BlockSpec((_FBLK, D), lambda i: (i, 0)),
        out_shape=jax.ShapeDtypeStruct((NNZ, D), jnp.float32),
    )(values, gg, gg, w0)


def kernel(values, row_idx, col_idx, W, bias):
    idxflat = jnp.concatenate([row_idx, col_idx])
    idxflatb = jnp.concatenate([row_idx, col_idx + N_SEG_P])
    zsum = jnp.zeros((N_SEG_P, D), jnp.float32)
    ones_in = jnp.ones((SUB, D), jnp.float32)
    sums = _seg_sums()(values, idxflat, zsum)
    cnts = _seg_cnts()(idxflat, zsum, ones_in)
    tabs = _tables(sums, cnts, W, bias)
    gg = _gather()(tabs.reshape(NC * N_SEG_P, D), idxflatb)
    return _final(values, gg, W[0])
